# Initial kernel scaffold; baseline (speedup 1.0000x reference)
import jax
import jax.numpy as jnp
from jax.experimental import pallas as pl


def kernel(pred_depth, gt_depth, tgt_valid_weight):
    raise NotImplementedError("write your pallas kernel here")


# speedup vs baseline: 1.9581x; 1.9581x over previous
import functools

import numpy as np
import jax
import jax.numpy as jnp
from jax import lax
from jax.experimental import pallas as pl
from jax.experimental.pallas import tpu as pltpu
from jax.experimental.pallas import tpu_sc as plsc

_B = 4
_HW = 262144
_K = 52428
_ROWS = 2048
_KP = 53248
_IVC = 3328
_PRC = 1664
_NPP = 26624

_rng = np.random.default_rng(0)
_samp = np.stack([_rng.integers(0, _HW - _K, size=_K) for _ in range(_B)]).astype(np.int32)
_mask_A = _rng.random(_HW) >= (1.0 - 0.1)
_perm = _rng.permutation(_HW)
_idx_A = np.flatnonzero(_mask_A).astype(np.int32)
_idx_B = np.flatnonzero(_mask_A[_perm]).astype(np.int32)
_NP = int(len(_idx_A))

_row_off = (np.arange(_B, dtype=np.int64) * _HW)[:, None].astype(np.int32)
_samp_p = np.zeros((_B, _KP), np.int32)
_samp_p[:, :_K] = _samp
_GIDX = (_samp_p + _K + _row_off).reshape(_B, 16, _IVC)
_pa = np.zeros(_NPP, np.int32)
_pa[:_NP] = _idx_A
_pb = np.zeros(_NPP, np.int32)
_pb[:_NP] = _idx_B
_PA4 = (_pa[None, :] + _row_off).reshape(_B, 16, _PRC)
_PB4 = (_pb[None, :] + _row_off).reshape(_B, 16, _PRC)


def _excl_prefix(x):
    i0 = lax.broadcasted_iota(jnp.int32, (128, 128), 0)
    i1 = lax.broadcasted_iota(jnp.int32, (128, 128), 1)
    a = (i0 < i1).astype(jnp.float32)
    within = jnp.dot(x, a, preferred_element_type=jnp.float32)
    s = jnp.sum(x, axis=1, keepdims=True)
    inc = s
    sh = 1
    while sh < _ROWS:
        z = jnp.zeros((sh, 1), jnp.float32)
        inc = inc + jnp.concatenate([z, inc[:-sh]], axis=0)
        sh *= 2
    return within + (inc - s)


def _dst_body(bits_ref, dst_ref):
    b = pl.program_id(0)
    x = bits_ref[0]
    kf = jnp.float32(_K)

    def it(i, p):
        cand = p | (jnp.int32(1) << (jnp.int32(29) - i))
        cnt = jnp.sum((x < cand).astype(jnp.float32))
        return jnp.where(cnt < kf, cand, p)

    t = lax.fori_loop(0, 30, it, jnp.int32(0))
    lt = x < t
    eq = x == t
    count_lt = jnp.sum(lt.astype(jnp.float32))
    c = kf - count_lt
    eq_rank = _excl_prefix(eq.astype(jnp.float32))
    is_inval = lt | (eq & (eq_rank < c))
    iv_rank = _excl_prefix(is_inval.astype(jnp.float32))
    rid = lax.broadcasted_iota(jnp.int32, (_ROWS, 128), 0)
    cid = lax.broadcasted_iota(jnp.int32, (_ROWS, 128), 1)
    pidx = (rid * 128 + cid).astype(jnp.float32)
    dstf = jnp.where(is_inval, iv_rank, kf + pidx - iv_rank)
    dst_ref[0] = dstf.astype(jnp.int32) + b * _HW


_dst_call = pl.pallas_call(
    _dst_body,
    grid=(_B,),
    in_specs=[pl.BlockSpec((1, _ROWS, 128), lambda b: (b, 0, 0))],
    out_specs=pl.BlockSpec((1, _ROWS, 128), lambda b: (b, 0, 0)),
    out_shape=jax.ShapeDtypeStruct((_B, _ROWS, 128), jnp.int32),
)


_CH = _HW // 32


def _sc_scatter_body(dst_hbm, iota_hbm, pos_hbm, dst_v, val_v, sem):
    c = lax.axis_index("c")
    s = lax.axis_index("s")
    wid = s * 2 + c
    e0 = wid * _CH
    for b in range(_B):
        pltpu.sync_copy(dst_hbm.at[b, pl.ds(e0, _CH)], dst_v)
        pltpu.sync_copy(iota_hbm.at[b, pl.ds(e0, _CH)], val_v)
        pltpu.async_copy(val_v, pos_hbm.at[dst_v], sem).wait()


def _sc_gather_body(pos_hbm, gidx_hbm, pa_hbm, pb_hbm, gt_hbm, pr_hbm,
               o_gtiv, o_priv, o_gtv, o_prv, o_ga, o_gb, o_pa, o_pb,
               iv_x, gx_v, vp_x, pa_x, pb_x,
               b_gtiv, b_priv, b_gtv, b_prv, b_ga, b_gb, b_pa, b_pb, sem):
    c = lax.axis_index("c")
    s = lax.axis_index("s")
    for bi in range(2):
        b = c * 2 + bi
        pltpu.sync_copy(pos_hbm.at[pl.ds(b * _HW + s * _IVC, _IVC)], iv_x)
        h1 = pltpu.async_copy(gt_hbm.at[iv_x], b_gtiv, sem)
        h2 = pltpu.async_copy(pr_hbm.at[iv_x], b_priv, sem)
        pltpu.sync_copy(gidx_hbm.at[b, s], gx_v)
        pltpu.async_copy(pos_hbm.at[gx_v], vp_x, sem).wait()
        h3 = pltpu.async_copy(gt_hbm.at[vp_x], b_gtv, sem)
        h4 = pltpu.async_copy(pr_hbm.at[vp_x], b_prv, sem)
        pltpu.sync_copy(pa_hbm.at[b, s], pa_x)
        pltpu.sync_copy(pb_hbm.at[b, s], pb_x)
        h5 = pltpu.async_copy(gt_hbm.at[pa_x], b_ga, sem)
        h6 = pltpu.async_copy(gt_hbm.at[pb_x], b_gb, sem)
        h7 = pltpu.async_copy(pr_hbm.at[pa_x], b_pa, sem)
        h8 = pltpu.async_copy(pr_hbm.at[pb_x], b_pb, sem)
        h1.wait()
        h2.wait()
        h3.wait()
        h4.wait()
        h5.wait()
        h6.wait()
        h7.wait()
        h8.wait()
        pltpu.sync_copy(b_gtiv, o_gtiv.at[b, s])
        pltpu.sync_copy(b_priv, o_priv.at[b, s])
        pltpu.sync_copy(b_gtv, o_gtv.at[b, s])
        pltpu.sync_copy(b_prv, o_prv.at[b, s])
        pltpu.sync_copy(b_ga, o_ga.at[b, s])
        pltpu.sync_copy(b_gb, o_gb.at[b, s])
        pltpu.sync_copy(b_pa, o_pa.at[b, s])
        pltpu.sync_copy(b_pb, o_pb.at[b, s])


@functools.lru_cache(maxsize=1)
def _sc_calls():
    mesh = plsc.VectorSubcoreMesh(core_axis_name="c", subcore_axis_name="s")
    scatter = pl.kernel(
        _sc_scatter_body,
        out_type=jax.ShapeDtypeStruct((_B * _HW,), jnp.int32),
        mesh=mesh,
        scratch_types=[
            pltpu.VMEM((_CH,), jnp.int32),
            pltpu.VMEM((_CH,), jnp.int32),
            pltpu.SemaphoreType.DMA,
        ],
    )
    f4 = jax.ShapeDtypeStruct((_B, 16, _IVC), jnp.float32)
    p4 = jax.ShapeDtypeStruct((_B, 16, _PRC), jnp.float32)
    gather = pl.kernel(
        _sc_gather_body,
        out_type=[f4, f4, f4, f4, p4, p4, p4, p4],
        mesh=mesh,
        scratch_types=[
            pltpu.VMEM((_IVC,), jnp.int32),
            pltpu.VMEM((_IVC,), jnp.int32),
            pltpu.VMEM((_IVC,), jnp.int32),
            pltpu.VMEM((_PRC,), jnp.int32),
            pltpu.VMEM((_PRC,), jnp.int32),
            pltpu.VMEM((_IVC,), jnp.float32),
            pltpu.VMEM((_IVC,), jnp.float32),
            pltpu.VMEM((_IVC,), jnp.float32),
            pltpu.VMEM((_IVC,), jnp.float32),
            pltpu.VMEM((_PRC,), jnp.float32),
            pltpu.VMEM((_PRC,), jnp.float32),
            pltpu.VMEM((_PRC,), jnp.float32),
            pltpu.VMEM((_PRC,), jnp.float32),
            pltpu.SemaphoreType.DMA,
        ],
    )
    return scatter, gather


def _loss_body(gtiv_ref, priv_ref, gtv_ref, prv_ref, ga_ref, gb_ref,
               pa_ref, pb_ref, out_ref):
    thr = jnp.float32(1.0 + 0.15)
    a = gtiv_ref[...]
    v = gtv_ref[...]
    m1 = lax.broadcasted_iota(jnp.int32, (_B, _KP), 1) < _K
    t1 = jnp.where(a / v >= thr, 1.0, jnp.where(v / a > thr, -1.0, 0.0)).astype(jnp.float32)
    w1 = (t1 != 0.0) & m1
    pd1 = priv_ref[...] - prv_ref[...]
    l1 = jnp.sum(jnp.where(w1, jnp.log(1.0 + jnp.exp(-t1 * pd1)), 0.0))
    n1 = jnp.sum(w1.astype(jnp.float32))

    za = ga_ref[...]
    zb = gb_ref[...]
    m2 = lax.broadcasted_iota(jnp.int32, (_B, _NPP), 1) < _NP
    ok = (za > 1e-8) | (zb > 1e-8)
    t2 = jnp.where(za / zb > thr, 1.0, jnp.where(zb / za > thr, -1.0, 0.0)).astype(jnp.float32)
    w2 = ok & (t2 != 0.0) & m2
    pd2 = pa_ref[...] - pb_ref[...]
    l2 = jnp.sum(jnp.where(w2, jnp.log(1.0 + jnp.exp(-t2 * pd2)), 0.0))
    n2 = jnp.sum(w2.astype(jnp.float32))
    out_ref[0, 0] = (l2 + l1) / (n2 + n1)


_loss_call = pl.pallas_call(
    _loss_body,
    out_specs=pl.BlockSpec(memory_space=pltpu.SMEM),
    out_shape=jax.ShapeDtypeStruct((1, 1), jnp.float32),
)


def kernel(pred_depth, gt_depth, tgt_valid_weight):
    bits = lax.bitcast_convert_type(tgt_valid_weight, jnp.int32).reshape(_B, _ROWS, 128)
    dst = _dst_call(bits).reshape(_B, _HW)
    iota = jnp.arange(_B * _HW, dtype=jnp.int32).reshape(_B, _HW)
    sc_scatter, sc_gather = _sc_calls()
    pos = sc_scatter(dst, iota)
    gt1 = gt_depth.reshape(-1)
    pr1 = pred_depth.reshape(-1)
    outs = sc_gather(pos, jnp.asarray(_GIDX), jnp.asarray(_PA4),
                     jnp.asarray(_PB4), gt1, pr1)
    gtiv, priv, gtv, prv = [o.reshape(_B, _KP) for o in outs[:4]]
    ga, gb, pa, pb = [o.reshape(_B, _NPP) for o in outs[4:]]
    total = _loss_call(gtiv, priv, gtv, prv, ga, gb, pa, pb)
    return total[0, 0]


# speedup vs baseline: 19.3536x; 9.8841x over previous
import functools

import numpy as np
import jax
import jax.numpy as jnp
from jax import lax
from jax.experimental import pallas as pl
from jax.experimental.pallas import tpu as pltpu
from jax.experimental.pallas import tpu_sc as plsc

_B = 4
_HW = 262144
_K = 52428
_ROWS = 2048
_KP = 53248
_IVC = 3328
_PRC = 1664
_NPP = 26624

_rng = np.random.default_rng(0)
_samp = np.stack([_rng.integers(0, _HW - _K, size=_K) for _ in range(_B)]).astype(np.int32)
_mask_A = _rng.random(_HW) >= (1.0 - 0.1)
_perm = _rng.permutation(_HW)
_idx_A = np.flatnonzero(_mask_A).astype(np.int32)
_idx_B = np.flatnonzero(_mask_A[_perm]).astype(np.int32)
_NP = int(len(_idx_A))

_row_off = (np.arange(_B, dtype=np.int64) * _HW)[:, None].astype(np.int32)
_samp_p = np.zeros((_B, _KP), np.int32)
_samp_p[:, :_K] = _samp
_GIDX = (_samp_p + _K + _row_off).reshape(_B, 16, _IVC)
_pa = np.zeros(_NPP, np.int32)
_pa[:_NP] = _idx_A
_pb = np.zeros(_NPP, np.int32)
_pb[:_NP] = _idx_B
_PA4 = (_pa[None, :] + _row_off).reshape(_B, 16, _PRC)
_PB4 = (_pb[None, :] + _row_off).reshape(_B, 16, _PRC)


def _excl_prefix(x):
    i0 = lax.broadcasted_iota(jnp.int32, (128, 128), 0)
    i1 = lax.broadcasted_iota(jnp.int32, (128, 128), 1)
    a = (i0 < i1).astype(jnp.float32)
    within = jnp.dot(x, a, preferred_element_type=jnp.float32)
    s = jnp.sum(x, axis=1, keepdims=True)
    inc = s
    sh = 1
    while sh < _ROWS:
        z = jnp.zeros((sh, 1), jnp.float32)
        inc = inc + jnp.concatenate([z, inc[:-sh]], axis=0)
        sh *= 2
    return within + (inc - s)


def _dst_body(bits_ref, dst_ref):
    b = pl.program_id(0)
    x = bits_ref[0]
    kf = jnp.float32(_K)

    def it(i, p):
        cand = p | (jnp.int32(1) << (jnp.int32(29) - i))
        cnt = jnp.sum((x < cand).astype(jnp.float32))
        return jnp.where(cnt < kf, cand, p)

    t = lax.fori_loop(0, 30, it, jnp.int32(0))
    lt = x < t
    eq = x == t
    count_lt = jnp.sum(lt.astype(jnp.float32))
    c = kf - count_lt
    eq_rank = _excl_prefix(eq.astype(jnp.float32))
    is_inval = lt | (eq & (eq_rank < c))
    iv_rank = _excl_prefix(is_inval.astype(jnp.float32))
    rid = lax.broadcasted_iota(jnp.int32, (_ROWS, 128), 0)
    cid = lax.broadcasted_iota(jnp.int32, (_ROWS, 128), 1)
    pidx = (rid * 128 + cid).astype(jnp.float32)
    dstf = jnp.where(is_inval, iv_rank, kf + pidx - iv_rank)
    dst_ref[0] = dstf.astype(jnp.int32) + (b % 2) * _HW


_dst_call = pl.pallas_call(
    _dst_body,
    grid=(_B,),
    in_specs=[pl.BlockSpec((1, _ROWS, 128), lambda b: (b, 0, 0))],
    out_specs=pl.BlockSpec((1, _ROWS, 128), lambda b: (b, 0, 0)),
    out_shape=jax.ShapeDtypeStruct((_B, _ROWS, 128), jnp.int32),
)


_CH = _HW // 16
_DR = _HW * 2 // 16


def _sc_scatter_body(dst_hbm, iota_hbm, pos_hbm, dst_v, val_v, pos_sh, sem):
    c = lax.axis_index("c")
    s = lax.axis_index("s")
    e0 = s * _CH
    for bi in range(2):
        b = c * 2 + bi
        pltpu.sync_copy(dst_hbm.at[b, pl.ds(e0, _CH)], dst_v)
        pltpu.sync_copy(iota_hbm.at[b, pl.ds(e0, _CH)], val_v)
        pltpu.async_copy(val_v, pos_sh.at[dst_v], sem).wait()
    plsc.subcore_barrier()
    off = s * _DR
    pltpu.sync_copy(pos_sh.at[pl.ds(off, _DR)],
                    pos_hbm.at[pl.ds(c * 2 * _HW + off, _DR)])


def _sc_gather_body(pos_hbm, gidx_hbm, pa_hbm, pb_hbm, gt_hbm, pr_hbm,
               o_gtiv, o_priv, o_gtv, o_prv, o_ga, o_gb, o_pa, o_pb,
               iv_x, gx_v, vp_x, pa_x, pb_x,
               b_gtiv, b_priv, b_gtv, b_prv, b_ga, b_gb, b_pa, b_pb, sem):
    c = lax.axis_index("c")
    s = lax.axis_index("s")
    for bi in range(2):
        b = c * 2 + bi
        pltpu.sync_copy(pos_hbm.at[pl.ds(b * _HW + s * _IVC, _IVC)], iv_x)
        h1 = pltpu.async_copy(gt_hbm.at[iv_x], b_gtiv, sem)
        h2 = pltpu.async_copy(pr_hbm.at[iv_x], b_priv, sem)
        pltpu.sync_copy(gidx_hbm.at[b, s], gx_v)
        pltpu.async_copy(pos_hbm.at[gx_v], vp_x, sem).wait()
        h3 = pltpu.async_copy(gt_hbm.at[vp_x], b_gtv, sem)
        h4 = pltpu.async_copy(pr_hbm.at[vp_x], b_prv, sem)
        pltpu.sync_copy(pa_hbm.at[b, s], pa_x)
        pltpu.sync_copy(pb_hbm.at[b, s], pb_x)
        h5 = pltpu.async_copy(gt_hbm.at[pa_x], b_ga, sem)
        h6 = pltpu.async_copy(gt_hbm.at[pb_x], b_gb, sem)
        h7 = pltpu.async_copy(pr_hbm.at[pa_x], b_pa, sem)
        h8 = pltpu.async_copy(pr_hbm.at[pb_x], b_pb, sem)
        h1.wait()
        h2.wait()
        h3.wait()
        h4.wait()
        h5.wait()
        h6.wait()
        h7.wait()
        h8.wait()
        pltpu.sync_copy(b_gtiv, o_gtiv.at[b, s])
        pltpu.sync_copy(b_priv, o_priv.at[b, s])
        pltpu.sync_copy(b_gtv, o_gtv.at[b, s])
        pltpu.sync_copy(b_prv, o_prv.at[b, s])
        pltpu.sync_copy(b_ga, o_ga.at[b, s])
        pltpu.sync_copy(b_gb, o_gb.at[b, s])
        pltpu.sync_copy(b_pa, o_pa.at[b, s])
        pltpu.sync_copy(b_pb, o_pb.at[b, s])


@functools.lru_cache(maxsize=1)
def _sc_calls():
    mesh = plsc.VectorSubcoreMesh(core_axis_name="c", subcore_axis_name="s")
    scatter = pl.kernel(
        _sc_scatter_body,
        out_type=jax.ShapeDtypeStruct((_B * _HW,), jnp.int32),
        mesh=mesh,
        scratch_types=[
            pltpu.VMEM((_CH,), jnp.int32),
            pltpu.VMEM((_CH,), jnp.int32),
            pltpu.VMEM_SHARED((2 * _HW,), jnp.int32),
            pltpu.SemaphoreType.DMA,
        ],
    )
    f4 = jax.ShapeDtypeStruct((_B, 16, _IVC), jnp.float32)
    p4 = jax.ShapeDtypeStruct((_B, 16, _PRC), jnp.float32)
    gather = pl.kernel(
        _sc_gather_body,
        out_type=[f4, f4, f4, f4, p4, p4, p4, p4],
        mesh=mesh,
        scratch_types=[
            pltpu.VMEM((_IVC,), jnp.int32),
            pltpu.VMEM((_IVC,), jnp.int32),
            pltpu.VMEM((_IVC,), jnp.int32),
            pltpu.VMEM((_PRC,), jnp.int32),
            pltpu.VMEM((_PRC,), jnp.int32),
            pltpu.VMEM((_IVC,), jnp.float32),
            pltpu.VMEM((_IVC,), jnp.float32),
            pltpu.VMEM((_IVC,), jnp.float32),
            pltpu.VMEM((_IVC,), jnp.float32),
            pltpu.VMEM((_PRC,), jnp.float32),
            pltpu.VMEM((_PRC,), jnp.float32),
            pltpu.VMEM((_PRC,), jnp.float32),
            pltpu.VMEM((_PRC,), jnp.float32),
            pltpu.SemaphoreType.DMA,
        ],
    )
    return scatter, gather


def _loss_body(gtiv_ref, priv_ref, gtv_ref, prv_ref, ga_ref, gb_ref,
               pa_ref, pb_ref, out_ref):
    thr = jnp.float32(1.0 + 0.15)
    a = gtiv_ref[...]
    v = gtv_ref[...]
    m1 = lax.broadcasted_iota(jnp.int32, (_B, _KP), 1) < _K
    t1 = jnp.where(a / v >= thr, 1.0, jnp.where(v / a > thr, -1.0, 0.0)).astype(jnp.float32)
    w1 = (t1 != 0.0) & m1
    pd1 = priv_ref[...] - prv_ref[...]
    l1 = jnp.sum(jnp.where(w1, jnp.log(1.0 + jnp.exp(-t1 * pd1)), 0.0))
    n1 = jnp.sum(w1.astype(jnp.float32))

    za = ga_ref[...]
    zb = gb_ref[...]
    m2 = lax.broadcasted_iota(jnp.int32, (_B, _NPP), 1) < _NP
    ok = (za > 1e-8) | (zb > 1e-8)
    t2 = jnp.where(za / zb > thr, 1.0, jnp.where(zb / za > thr, -1.0, 0.0)).astype(jnp.float32)
    w2 = ok & (t2 != 0.0) & m2
    pd2 = pa_ref[...] - pb_ref[...]
    l2 = jnp.sum(jnp.where(w2, jnp.log(1.0 + jnp.exp(-t2 * pd2)), 0.0))
    n2 = jnp.sum(w2.astype(jnp.float32))
    out_ref[0, 0] = (l2 + l1) / (n2 + n1)


_loss_call = pl.pallas_call(
    _loss_body,
    out_specs=pl.BlockSpec(memory_space=pltpu.SMEM),
    out_shape=jax.ShapeDtypeStruct((1, 1), jnp.float32),
)


def kernel(pred_depth, gt_depth, tgt_valid_weight):
    bits = lax.bitcast_convert_type(tgt_valid_weight, jnp.int32).reshape(_B, _ROWS, 128)
    dst = _dst_call(bits).reshape(_B, _HW)
    iota = jnp.arange(_B * _HW, dtype=jnp.int32).reshape(_B, _HW)
    sc_scatter, sc_gather = _sc_calls()
    pos = sc_scatter(dst, iota)
    gt1 = gt_depth.reshape(-1)
    pr1 = pred_depth.reshape(-1)
    outs = sc_gather(pos, jnp.asarray(_GIDX), jnp.asarray(_PA4),
                     jnp.asarray(_PB4), gt1, pr1)
    gtiv, priv, gtv, prv = [o.reshape(_B, _KP) for o in outs[:4]]
    ga, gb, pa, pb = [o.reshape(_B, _NPP) for o in outs[4:]]
    total = _loss_call(gtiv, priv, gtv, prv, ga, gb, pa, pb)
    return total[0, 0]


# speedup vs baseline: 26.8273x; 1.3862x over previous
import functools

import numpy as np
import jax
import jax.numpy as jnp
from jax import lax
from jax.experimental import pallas as pl
from jax.experimental.pallas import tpu as pltpu
from jax.experimental.pallas import tpu_sc as plsc

_B = 4
_HW = 262144
_K = 52428
_ROWS = 2048
_KP = 53248
_IVC = 3328
_PRC = 1664
_NPP = 26624

_rng = np.random.default_rng(0)
_samp = np.stack([_rng.integers(0, _HW - _K, size=_K) for _ in range(_B)]).astype(np.int32)
_mask_A = _rng.random(_HW) >= (1.0 - 0.1)
_perm = _rng.permutation(_HW)
_idx_A = np.flatnonzero(_mask_A).astype(np.int32)
_idx_B = np.flatnonzero(_mask_A[_perm]).astype(np.int32)
_NP = int(len(_idx_A))

_glob_off = (np.arange(_B)[:, None] * _HW).astype(np.int32)
_samp_p = np.zeros((_B, _KP), np.int32)
_samp_p[:, :_K] = _samp
_GIDX = (_samp_p + _K).reshape(-1)
_pa = np.zeros(_NPP, np.int32)
_pa[:_NP] = _idx_A
_pb = np.zeros(_NPP, np.int32)
_pb[:_NP] = _idx_B
_PA4 = (_pa[None, :] + _glob_off).reshape(-1)
_PB4 = (_pb[None, :] + _glob_off).reshape(-1)


def _excl_prefix(x):
    i0 = lax.broadcasted_iota(jnp.int32, (128, 128), 0)
    i1 = lax.broadcasted_iota(jnp.int32, (128, 128), 1)
    a = (i0 < i1).astype(jnp.float32)
    within = jnp.dot(x, a, preferred_element_type=jnp.float32)
    s = jnp.sum(x, axis=1, keepdims=True)
    inc = s
    sh = 1
    while sh < _ROWS:
        z = jnp.zeros((sh, 1), jnp.float32)
        inc = inc + jnp.concatenate([z, inc[:-sh]], axis=0)
        sh *= 2
    return within + (inc - s)


def _dst_body(bits_ref, dst_ref):
    x = bits_ref[0]
    kf = jnp.float32(_K)

    def it(i, p):
        cand = p | (jnp.int32(1) << (jnp.int32(29) - i))
        cnt = jnp.sum((x < cand).astype(jnp.float32))
        return jnp.where(cnt < kf, cand, p)

    t = lax.fori_loop(0, 30, it, jnp.int32(0))
    lt = x < t
    eq = x == t
    count_lt = jnp.sum(lt.astype(jnp.float32))
    c = kf - count_lt
    eq_rank = _excl_prefix(eq.astype(jnp.float32))
    is_inval = lt | (eq & (eq_rank < c))
    iv_rank = _excl_prefix(is_inval.astype(jnp.float32))
    rid = lax.broadcasted_iota(jnp.int32, (_ROWS, 128), 0)
    cid = lax.broadcasted_iota(jnp.int32, (_ROWS, 128), 1)
    pidx = (rid * 128 + cid).astype(jnp.float32)
    dstf = jnp.where(is_inval, iv_rank, kf + pidx - iv_rank)
    dst_ref[0] = dstf.astype(jnp.int32)


_dst_call = pl.pallas_call(
    _dst_body,
    grid=(_B,),
    in_specs=[pl.BlockSpec((1, _ROWS, 128), lambda b: (b, 0, 0))],
    out_specs=pl.BlockSpec((1, _ROWS, 128), lambda b: (b, 0, 0)),
    out_shape=jax.ShapeDtypeStruct((_B, _ROWS, 128), jnp.int32),
)


_CH = _HW // 16


def _sc_body(dst_hbm, gidx_hbm, pa_hbm, pb_hbm, gt_hbm, pr_hbm,
             o_gtiv, o_priv, o_gtv, o_prv, o_ga, o_gb, o_pa, o_pb,
             dst_v, gval, pval, gx_v, pa_x, pb_x,
             b_gtv, b_prv, b_ga, b_gb, b_pa, b_pb,
             gtc_sh, prc_sh, sem, semp):
    c = lax.axis_index("c")
    s = lax.axis_index("s")
    e0 = s * _CH
    for bi in range(2):
        b = c * 2 + bi
        base = b * _HW + e0
        oo_p = (b * 16 + s) * _PRC
        oo = (b * 16 + s) * _IVC
        pltpu.sync_copy(pa_hbm.at[pl.ds(oo_p, _PRC)], pa_x)
        pltpu.sync_copy(pb_hbm.at[pl.ds(oo_p, _PRC)], pb_x)
        hp1 = pltpu.async_copy(gt_hbm.at[pa_x], b_ga, semp)
        hp2 = pltpu.async_copy(gt_hbm.at[pb_x], b_gb, semp)
        hp3 = pltpu.async_copy(pr_hbm.at[pa_x], b_pa, semp)
        hp4 = pltpu.async_copy(pr_hbm.at[pb_x], b_pb, semp)
        pltpu.sync_copy(dst_hbm.at[pl.ds(base, _CH)], dst_v)
        pltpu.sync_copy(gt_hbm.at[pl.ds(base, _CH)], gval)
        pltpu.sync_copy(pr_hbm.at[pl.ds(base, _CH)], pval)
        h1 = pltpu.async_copy(gval, gtc_sh.at[dst_v], sem)
        h2 = pltpu.async_copy(pval, prc_sh.at[dst_v], sem)
        h1.wait()
        h2.wait()
        hp1.wait()
        hp2.wait()
        hp3.wait()
        hp4.wait()
        pltpu.sync_copy(b_ga, o_ga.at[pl.ds(oo_p, _PRC)])
        pltpu.sync_copy(b_gb, o_gb.at[pl.ds(oo_p, _PRC)])
        pltpu.sync_copy(b_pa, o_pa.at[pl.ds(oo_p, _PRC)])
        pltpu.sync_copy(b_pb, o_pb.at[pl.ds(oo_p, _PRC)])
        plsc.subcore_barrier()
        lo = s * _IVC
        pltpu.sync_copy(gtc_sh.at[pl.ds(lo, _IVC)], o_gtiv.at[pl.ds(oo, _IVC)])
        pltpu.sync_copy(prc_sh.at[pl.ds(lo, _IVC)], o_priv.at[pl.ds(oo, _IVC)])
        pltpu.sync_copy(gidx_hbm.at[pl.ds(oo, _IVC)], gx_v)
        h3 = pltpu.async_copy(gtc_sh.at[gx_v], b_gtv, sem)
        h4 = pltpu.async_copy(prc_sh.at[gx_v], b_prv, sem)
        h3.wait()
        h4.wait()
        pltpu.sync_copy(b_gtv, o_gtv.at[pl.ds(oo, _IVC)])
        pltpu.sync_copy(b_prv, o_prv.at[pl.ds(oo, _IVC)])
        plsc.subcore_barrier()


@functools.lru_cache(maxsize=1)
def _sc_calls():
    mesh = plsc.VectorSubcoreMesh(core_axis_name="c", subcore_axis_name="s")
    f4 = jax.ShapeDtypeStruct((_B * 16 * _IVC,), jnp.float32)
    p4 = jax.ShapeDtypeStruct((_B * 16 * _PRC,), jnp.float32)
    return pl.kernel(
        _sc_body,
        out_type=[f4, f4, f4, f4, p4, p4, p4, p4],
        mesh=mesh,
        scratch_types=[
            pltpu.VMEM((_CH,), jnp.int32),
            pltpu.VMEM((_CH,), jnp.float32),
            pltpu.VMEM((_CH,), jnp.float32),
            pltpu.VMEM((_IVC,), jnp.int32),
            pltpu.VMEM((_PRC,), jnp.int32),
            pltpu.VMEM((_PRC,), jnp.int32),
            pltpu.VMEM((_IVC,), jnp.float32),
            pltpu.VMEM((_IVC,), jnp.float32),
            pltpu.VMEM((_PRC,), jnp.float32),
            pltpu.VMEM((_PRC,), jnp.float32),
            pltpu.VMEM((_PRC,), jnp.float32),
            pltpu.VMEM((_PRC,), jnp.float32),
            pltpu.VMEM_SHARED((_HW,), jnp.float32),
            pltpu.VMEM_SHARED((_HW,), jnp.float32),
            pltpu.SemaphoreType.DMA,
            pltpu.SemaphoreType.DMA,
        ],
    )


def _loss_body(gtiv_ref, priv_ref, gtv_ref, prv_ref, ga_ref, gb_ref,
               pa_ref, pb_ref, out_ref):
    thr = jnp.float32(1.0 + 0.15)
    a = gtiv_ref[...]
    v = gtv_ref[...]
    m1 = lax.broadcasted_iota(jnp.int32, (_B, _KP), 1) < _K
    t1 = jnp.where(a / v >= thr, 1.0, jnp.where(v / a > thr, -1.0, 0.0)).astype(jnp.float32)
    w1 = (t1 != 0.0) & m1
    pd1 = priv_ref[...] - prv_ref[...]
    l1 = jnp.sum(jnp.where(w1, jnp.log(1.0 + jnp.exp(-t1 * pd1)), 0.0))
    n1 = jnp.sum(w1.astype(jnp.float32))

    za = ga_ref[...]
    zb = gb_ref[...]
    m2 = lax.broadcasted_iota(jnp.int32, (_B, _NPP), 1) < _NP
    ok = (za > 1e-8) | (zb > 1e-8)
    t2 = jnp.where(za / zb > thr, 1.0, jnp.where(zb / za > thr, -1.0, 0.0)).astype(jnp.float32)
    w2 = ok & (t2 != 0.0) & m2
    pd2 = pa_ref[...] - pb_ref[...]
    l2 = jnp.sum(jnp.where(w2, jnp.log(1.0 + jnp.exp(-t2 * pd2)), 0.0))
    n2 = jnp.sum(w2.astype(jnp.float32))
    out_ref[0, 0] = (l2 + l1) / (n2 + n1)


_loss_call = pl.pallas_call(
    _loss_body,
    out_specs=pl.BlockSpec(memory_space=pltpu.SMEM),
    out_shape=jax.ShapeDtypeStruct((1, 1), jnp.float32),
)


def kernel(pred_depth, gt_depth, tgt_valid_weight):
    bits = lax.bitcast_convert_type(tgt_valid_weight, jnp.int32).reshape(_B, _ROWS, 128)
    dst = _dst_call(bits).reshape(_B * _HW)
    gt1 = gt_depth.reshape(-1)
    pr1 = pred_depth.reshape(-1)
    outs = _sc_calls()(dst, jnp.asarray(_GIDX), jnp.asarray(_PA4),
                       jnp.asarray(_PB4), gt1, pr1)
    gtiv, priv, gtv, prv = [o.reshape(_B, _KP) for o in outs[:4]]
    ga, gb, pa, pb = [o.reshape(_B, _NPP) for o in outs[4:]]
    total = _loss_call(gtiv, priv, gtv, prv, ga, gb, pa, pb)
    return total[0, 0]


# speedup vs baseline: 33.0527x; 1.2321x over previous
import functools

import numpy as np
import jax
import jax.numpy as jnp
from jax import lax
from jax.experimental import pallas as pl
from jax.experimental.pallas import tpu as pltpu
from jax.experimental.pallas import tpu_sc as plsc

_B = 4
_HW = 262144
_K = 52428
_ROWS = 2048
_KP = 53248
_IVC = 3328
_PRC = 1664
_NPP = 26624

_rng = np.random.default_rng(0)
_samp = np.stack([_rng.integers(0, _HW - _K, size=_K) for _ in range(_B)]).astype(np.int32)
_mask_A = _rng.random(_HW) >= (1.0 - 0.1)
_perm = _rng.permutation(_HW)
_idx_A = np.flatnonzero(_mask_A).astype(np.int32)
_idx_B = np.flatnonzero(_mask_A[_perm]).astype(np.int32)
_NP = int(len(_idx_A))

_glob_off = (np.arange(_B)[:, None] * _HW).astype(np.int32)
_samp_p = np.zeros((_B, _KP), np.int32)
_samp_p[:, :_K] = _samp
_GIDX = (_samp_p + _K).reshape(-1)
_pa = np.zeros(_NPP, np.int32)
_pa[:_NP] = _idx_A
_pb = np.zeros(_NPP, np.int32)
_pb[:_NP] = _idx_B
_PA4 = (_pa[None, :] + _glob_off).reshape(-1)
_PB4 = (_pb[None, :] + _glob_off).reshape(-1)


def _excl_prefix(x):
    i0 = lax.broadcasted_iota(jnp.int32, (128, 128), 0)
    i1 = lax.broadcasted_iota(jnp.int32, (128, 128), 1)
    a = (i0 < i1).astype(jnp.float32)
    within = jnp.dot(x, a, preferred_element_type=jnp.float32)
    s = jnp.sum(x, axis=1, keepdims=True)
    inc = s
    sh = 1
    while sh < _ROWS:
        z = jnp.zeros((sh, 1), jnp.float32)
        inc = inc + jnp.concatenate([z, inc[:-sh]], axis=0)
        sh *= 2
    return within + (inc - s)


def _thr_body(bits_ref, t_ref, c_ref):
    x = bits_ref[...]
    kf = jnp.float32(_K)

    def it(i, p):
        cand = p | (jnp.int32(1) << (jnp.int32(29) - i))
        cnt = jnp.sum((x < cand).astype(jnp.float32), axis=(1, 2), keepdims=True)
        return jnp.where(cnt < kf, cand, p)

    p = lax.fori_loop(0, 30, it, jnp.zeros((_B, 1, 1), jnp.int32))
    count_lt = jnp.sum((x < p).astype(jnp.float32), axis=(1, 2), keepdims=True)
    c = kf - count_lt
    t_ref[...] = jnp.broadcast_to(p, (_B, 1, 128))
    c_ref[...] = jnp.broadcast_to(c, (_B, 1, 128))


_thr_call = pl.pallas_call(
    _thr_body,
    out_shape=[jax.ShapeDtypeStruct((_B, 1, 128), jnp.int32),
               jax.ShapeDtypeStruct((_B, 1, 128), jnp.float32)],
)


def _dst_body(bits_ref, t_ref, c_ref, dst_ref):
    x = bits_ref[0]
    t = t_ref[0]
    c = c_ref[0]
    kf = jnp.float32(_K)
    lt = x < t
    eq = x == t
    eq_rank = _excl_prefix(eq.astype(jnp.float32))
    is_inval = lt | (eq & (eq_rank < c))
    iv_rank = _excl_prefix(is_inval.astype(jnp.float32))
    rid = lax.broadcasted_iota(jnp.int32, (_ROWS, 128), 0)
    cid = lax.broadcasted_iota(jnp.int32, (_ROWS, 128), 1)
    pidx = (rid * 128 + cid).astype(jnp.float32)
    dstf = jnp.where(is_inval, iv_rank, kf + pidx - iv_rank)
    dst_ref[0] = dstf.astype(jnp.int32)


_dst_call = pl.pallas_call(
    _dst_body,
    grid=(_B,),
    in_specs=[pl.BlockSpec((1, _ROWS, 128), lambda b: (b, 0, 0)),
              pl.BlockSpec((1, 1, 128), lambda b: (b, 0, 0)),
              pl.BlockSpec((1, 1, 128), lambda b: (b, 0, 0))],
    out_specs=pl.BlockSpec((1, _ROWS, 128), lambda b: (b, 0, 0)),
    out_shape=jax.ShapeDtypeStruct((_B, _ROWS, 128), jnp.int32),
)


_CH = _HW // 16


def _sc_body(dst_hbm, gidx_hbm, pa_hbm, pb_hbm, gt_hbm, pr_hbm,
             o_gtiv, o_priv, o_gtv, o_prv, o_ga, o_gb, o_pa, o_pb,
             dst_v, gval, pval, gx_v, pa_x, pb_x,
             b_gtv, b_prv, b_ga, b_gb, b_pa, b_pb,
             gtc_sh, prc_sh, sem, semp):
    c = lax.axis_index("c")
    s = lax.axis_index("s")
    e0 = s * _CH
    for bi in range(2):
        b = c * 2 + bi
        base = b * _HW + e0
        oo_p = (b * 16 + s) * _PRC
        oo = (b * 16 + s) * _IVC
        pltpu.sync_copy(pa_hbm.at[pl.ds(oo_p, _PRC)], pa_x)
        pltpu.sync_copy(pb_hbm.at[pl.ds(oo_p, _PRC)], pb_x)
        hp1 = pltpu.async_copy(gt_hbm.at[pa_x], b_ga, semp)
        hp2 = pltpu.async_copy(gt_hbm.at[pb_x], b_gb, semp)
        hp3 = pltpu.async_copy(pr_hbm.at[pa_x], b_pa, semp)
        hp4 = pltpu.async_copy(pr_hbm.at[pb_x], b_pb, semp)
        pltpu.sync_copy(dst_hbm.at[pl.ds(base, _CH)], dst_v)
        pltpu.sync_copy(gt_hbm.at[pl.ds(base, _CH)], gval)
        pltpu.sync_copy(pr_hbm.at[pl.ds(base, _CH)], pval)
        h1 = pltpu.async_copy(gval, gtc_sh.at[dst_v], sem)
        h2 = pltpu.async_copy(pval, prc_sh.at[dst_v], sem)
        h1.wait()
        h2.wait()
        hp1.wait()
        hp2.wait()
        hp3.wait()
        hp4.wait()
        pltpu.sync_copy(b_ga, o_ga.at[pl.ds(oo_p, _PRC)])
        pltpu.sync_copy(b_gb, o_gb.at[pl.ds(oo_p, _PRC)])
        pltpu.sync_copy(b_pa, o_pa.at[pl.ds(oo_p, _PRC)])
        pltpu.sync_copy(b_pb, o_pb.at[pl.ds(oo_p, _PRC)])
        plsc.subcore_barrier()
        lo = s * _IVC
        pltpu.sync_copy(gtc_sh.at[pl.ds(lo, _IVC)], o_gtiv.at[pl.ds(oo, _IVC)])
        pltpu.sync_copy(prc_sh.at[pl.ds(lo, _IVC)], o_priv.at[pl.ds(oo, _IVC)])
        pltpu.sync_copy(gidx_hbm.at[pl.ds(oo, _IVC)], gx_v)
        h3 = pltpu.async_copy(gtc_sh.at[gx_v], b_gtv, sem)
        h4 = pltpu.async_copy(prc_sh.at[gx_v], b_prv, sem)
        h3.wait()
        h4.wait()
        pltpu.sync_copy(b_gtv, o_gtv.at[pl.ds(oo, _IVC)])
        pltpu.sync_copy(b_prv, o_prv.at[pl.ds(oo, _IVC)])
        plsc.subcore_barrier()


@functools.lru_cache(maxsize=1)
def _sc_calls():
    mesh = plsc.VectorSubcoreMesh(core_axis_name="c", subcore_axis_name="s")
    f4 = jax.ShapeDtypeStruct((_B * 16 * _IVC,), jnp.float32)
    p4 = jax.ShapeDtypeStruct((_B * 16 * _PRC,), jnp.float32)
    return pl.kernel(
        _sc_body,
        out_type=[f4, f4, f4, f4, p4, p4, p4, p4],
        mesh=mesh,
        scratch_types=[
            pltpu.VMEM((_CH,), jnp.int32),
            pltpu.VMEM((_CH,), jnp.float32),
            pltpu.VMEM((_CH,), jnp.float32),
            pltpu.VMEM((_IVC,), jnp.int32),
            pltpu.VMEM((_PRC,), jnp.int32),
            pltpu.VMEM((_PRC,), jnp.int32),
            pltpu.VMEM((_IVC,), jnp.float32),
            pltpu.VMEM((_IVC,), jnp.float32),
            pltpu.VMEM((_PRC,), jnp.float32),
            pltpu.VMEM((_PRC,), jnp.float32),
            pltpu.VMEM((_PRC,), jnp.float32),
            pltpu.VMEM((_PRC,), jnp.float32),
            pltpu.VMEM_SHARED((_HW,), jnp.float32),
            pltpu.VMEM_SHARED((_HW,), jnp.float32),
            pltpu.SemaphoreType.DMA,
            pltpu.SemaphoreType.DMA,
        ],
    )


def _loss_body(gtiv_ref, priv_ref, gtv_ref, prv_ref, ga_ref, gb_ref,
               pa_ref, pb_ref, out_ref):
    thr = jnp.float32(1.0 + 0.15)
    a = gtiv_ref[...]
    v = gtv_ref[...]
    m1 = lax.broadcasted_iota(jnp.int32, (_B, _KP), 1) < _K
    t1 = jnp.where(a / v >= thr, 1.0, jnp.where(v / a > thr, -1.0, 0.0)).astype(jnp.float32)
    w1 = (t1 != 0.0) & m1
    pd1 = priv_ref[...] - prv_ref[...]
    l1 = jnp.sum(jnp.where(w1, jnp.log(1.0 + jnp.exp(-t1 * pd1)), 0.0))
    n1 = jnp.sum(w1.astype(jnp.float32))

    za = ga_ref[...]
    zb = gb_ref[...]
    m2 = lax.broadcasted_iota(jnp.int32, (_B, _NPP), 1) < _NP
    ok = (za > 1e-8) | (zb > 1e-8)
    t2 = jnp.where(za / zb > thr, 1.0, jnp.where(zb / za > thr, -1.0, 0.0)).astype(jnp.float32)
    w2 = ok & (t2 != 0.0) & m2
    pd2 = pa_ref[...] - pb_ref[...]
    l2 = jnp.sum(jnp.where(w2, jnp.log(1.0 + jnp.exp(-t2 * pd2)), 0.0))
    n2 = jnp.sum(w2.astype(jnp.float32))
    out_ref[0, 0] = (l2 + l1) / (n2 + n1)


_loss_call = pl.pallas_call(
    _loss_body,
    out_specs=pl.BlockSpec(memory_space=pltpu.SMEM),
    out_shape=jax.ShapeDtypeStruct((1, 1), jnp.float32),
)


def kernel(pred_depth, gt_depth, tgt_valid_weight):
    bits = lax.bitcast_convert_type(tgt_valid_weight, jnp.int32).reshape(_B, _ROWS, 128)
    thr, tie = _thr_call(bits)
    dst = _dst_call(bits, thr, tie).reshape(_B * _HW)
    gt1 = gt_depth.reshape(-1)
    pr1 = pred_depth.reshape(-1)
    outs = _sc_calls()(dst, jnp.asarray(_GIDX), jnp.asarray(_PA4),
                       jnp.asarray(_PB4), gt1, pr1)
    gtiv, priv, gtv, prv = [o.reshape(_B, _KP) for o in outs[:4]]
    ga, gb, pa, pb = [o.reshape(_B, _NPP) for o in outs[4:]]
    total = _loss_call(gtiv, priv, gtv, prv, ga, gb, pa, pb)
    return total[0, 0]


# speedup vs baseline: 33.4279x; 1.0114x over previous
import functools

import numpy as np
import jax
import jax.numpy as jnp
from jax import lax
from jax.experimental import pallas as pl
from jax.experimental.pallas import tpu as pltpu
from jax.experimental.pallas import tpu_sc as plsc

_B = 4
_HW = 262144
_K = 52428
_ROWS = 2048
_KP = 53248
_IVC = 3328
_PRC = 1664
_NPP = 26624

_rng = np.random.default_rng(0)
_samp = np.stack([_rng.integers(0, _HW - _K, size=_K) for _ in range(_B)]).astype(np.int32)
_mask_A = _rng.random(_HW) >= (1.0 - 0.1)
_perm = _rng.permutation(_HW)
_idx_A = np.flatnonzero(_mask_A).astype(np.int32)
_idx_B = np.flatnonzero(_mask_A[_perm]).astype(np.int32)
_NP = int(len(_idx_A))

_glob_off = (np.arange(_B)[:, None] * _HW).astype(np.int32)
_samp_p = np.zeros((_B, _KP), np.int32)
_samp_p[:, :_K] = _samp
_GIDX = (_samp_p + _K).reshape(-1)
_pa = np.zeros(_NPP, np.int32)
_pa[:_NP] = _idx_A
_pb = np.zeros(_NPP, np.int32)
_pb[:_NP] = _idx_B
_PA4 = (_pa[None, :] + _glob_off).reshape(-1)
_PB4 = (_pb[None, :] + _glob_off).reshape(-1)


def _excl_prefix(x):
    i0 = lax.broadcasted_iota(jnp.int32, (128, 128), 0)
    i1 = lax.broadcasted_iota(jnp.int32, (128, 128), 1)
    a = (i0 < i1).astype(jnp.float32)
    within = jnp.dot(x.reshape(_B * _ROWS, 128), a,
                     preferred_element_type=jnp.float32).reshape(_B, _ROWS, 128)
    s = jnp.sum(x, axis=2, keepdims=True)
    inc = s
    sh = 1
    while sh < _ROWS:
        z = jnp.zeros((_B, sh, 1), jnp.float32)
        inc = inc + jnp.concatenate([z, inc[:, :-sh]], axis=1)
        sh *= 2
    return within + (inc - s)


def _dst_body(bits_ref, dst_ref):
    x = bits_ref[...]
    kf = jnp.float32(_K)

    def it(i, p):
        cand = p | (jnp.int32(1) << (jnp.int32(29) - i))
        cnt = jnp.sum((x < cand).astype(jnp.float32), axis=(1, 2), keepdims=True)
        return jnp.where(cnt < kf, cand, p)

    p = lax.fori_loop(0, 30, it, jnp.zeros((_B, 1, 1), jnp.int32))
    count_lt = jnp.sum((x < p).astype(jnp.float32), axis=(1, 2), keepdims=True)
    c = kf - count_lt
    lt = x < p
    eq = x == p
    eq_rank = _excl_prefix(eq.astype(jnp.float32))
    is_inval = lt | (eq & (eq_rank < c))
    iv_rank = _excl_prefix(is_inval.astype(jnp.float32))
    rid = lax.broadcasted_iota(jnp.int32, (_ROWS, 128), 0)
    cid = lax.broadcasted_iota(jnp.int32, (_ROWS, 128), 1)
    pidx = (rid * 128 + cid).astype(jnp.float32)
    dstf = jnp.where(is_inval, iv_rank, kf + pidx - iv_rank)
    dst_ref[...] = dstf.astype(jnp.int32)


_dst_call = pl.pallas_call(
    _dst_body,
    out_shape=jax.ShapeDtypeStruct((_B, _ROWS, 128), jnp.int32),
)


_CH = _HW // 16


def _sc_body(dst_hbm, gidx_hbm, pa_hbm, pb_hbm, gt_hbm, pr_hbm,
             o_gtiv, o_priv, o_gtv, o_prv, o_ga, o_gb, o_pa, o_pb,
             dst_v, gval, pval, gx_v, pa_x, pb_x,
             b_gtv, b_prv, b_ga, b_gb, b_pa, b_pb,
             gtc_sh, prc_sh, sem, semp):
    c = lax.axis_index("c")
    s = lax.axis_index("s")
    e0 = s * _CH
    for bi in range(2):
        b = c * 2 + bi
        base = b * _HW + e0
        oo_p = (b * 16 + s) * _PRC
        oo = (b * 16 + s) * _IVC
        pltpu.sync_copy(pa_hbm.at[pl.ds(oo_p, _PRC)], pa_x)
        pltpu.sync_copy(pb_hbm.at[pl.ds(oo_p, _PRC)], pb_x)
        hp1 = pltpu.async_copy(gt_hbm.at[pa_x], b_ga, semp)
        hp2 = pltpu.async_copy(gt_hbm.at[pb_x], b_gb, semp)
        hp3 = pltpu.async_copy(pr_hbm.at[pa_x], b_pa, semp)
        hp4 = pltpu.async_copy(pr_hbm.at[pb_x], b_pb, semp)
        pltpu.sync_copy(dst_hbm.at[pl.ds(base, _CH)], dst_v)
        pltpu.sync_copy(gt_hbm.at[pl.ds(base, _CH)], gval)
        pltpu.sync_copy(pr_hbm.at[pl.ds(base, _CH)], pval)
        h1 = pltpu.async_copy(gval, gtc_sh.at[dst_v], sem)
        h2 = pltpu.async_copy(pval, prc_sh.at[dst_v], sem)
        h1.wait()
        h2.wait()
        hp1.wait()
        hp2.wait()
        hp3.wait()
        hp4.wait()
        pltpu.sync_copy(b_ga, o_ga.at[pl.ds(oo_p, _PRC)])
        pltpu.sync_copy(b_gb, o_gb.at[pl.ds(oo_p, _PRC)])
        pltpu.sync_copy(b_pa, o_pa.at[pl.ds(oo_p, _PRC)])
        pltpu.sync_copy(b_pb, o_pb.at[pl.ds(oo_p, _PRC)])
        plsc.subcore_barrier()
        lo = s * _IVC
        pltpu.sync_copy(gtc_sh.at[pl.ds(lo, _IVC)], o_gtiv.at[pl.ds(oo, _IVC)])
        pltpu.sync_copy(prc_sh.at[pl.ds(lo, _IVC)], o_priv.at[pl.ds(oo, _IVC)])
        pltpu.sync_copy(gidx_hbm.at[pl.ds(oo, _IVC)], gx_v)
        h3 = pltpu.async_copy(gtc_sh.at[gx_v], b_gtv, sem)
        h4 = pltpu.async_copy(prc_sh.at[gx_v], b_prv, sem)
        h3.wait()
        h4.wait()
        pltpu.sync_copy(b_gtv, o_gtv.at[pl.ds(oo, _IVC)])
        pltpu.sync_copy(b_prv, o_prv.at[pl.ds(oo, _IVC)])
        plsc.subcore_barrier()


@functools.lru_cache(maxsize=1)
def _sc_calls():
    mesh = plsc.VectorSubcoreMesh(core_axis_name="c", subcore_axis_name="s")
    f4 = jax.ShapeDtypeStruct((_B * 16 * _IVC,), jnp.float32)
    p4 = jax.ShapeDtypeStruct((_B * 16 * _PRC,), jnp.float32)
    return pl.kernel(
        _sc_body,
        out_type=[f4, f4, f4, f4, p4, p4, p4, p4],
        mesh=mesh,
        scratch_types=[
            pltpu.VMEM((_CH,), jnp.int32),
            pltpu.VMEM((_CH,), jnp.float32),
            pltpu.VMEM((_CH,), jnp.float32),
            pltpu.VMEM((_IVC,), jnp.int32),
            pltpu.VMEM((_PRC,), jnp.int32),
            pltpu.VMEM((_PRC,), jnp.int32),
            pltpu.VMEM((_IVC,), jnp.float32),
            pltpu.VMEM((_IVC,), jnp.float32),
            pltpu.VMEM((_PRC,), jnp.float32),
            pltpu.VMEM((_PRC,), jnp.float32),
            pltpu.VMEM((_PRC,), jnp.float32),
            pltpu.VMEM((_PRC,), jnp.float32),
            pltpu.VMEM_SHARED((_HW,), jnp.float32),
            pltpu.VMEM_SHARED((_HW,), jnp.float32),
            pltpu.SemaphoreType.DMA,
            pltpu.SemaphoreType.DMA,
        ],
    )


def _loss_body(gtiv_ref, priv_ref, gtv_ref, prv_ref, ga_ref, gb_ref,
               pa_ref, pb_ref, out_ref):
    thr = jnp.float32(1.0 + 0.15)
    a = gtiv_ref[...]
    v = gtv_ref[...]
    m1 = lax.broadcasted_iota(jnp.int32, (_B, _KP), 1) < _K
    t1 = jnp.where(a / v >= thr, 1.0, jnp.where(v / a > thr, -1.0, 0.0)).astype(jnp.float32)
    w1 = (t1 != 0.0) & m1
    pd1 = priv_ref[...] - prv_ref[...]
    l1 = jnp.sum(jnp.where(w1, jnp.log(1.0 + jnp.exp(-t1 * pd1)), 0.0))
    n1 = jnp.sum(w1.astype(jnp.float32))

    za = ga_ref[...]
    zb = gb_ref[...]
    m2 = lax.broadcasted_iota(jnp.int32, (_B, _NPP), 1) < _NP
    ok = (za > 1e-8) | (zb > 1e-8)
    t2 = jnp.where(za / zb > thr, 1.0, jnp.where(zb / za > thr, -1.0, 0.0)).astype(jnp.float32)
    w2 = ok & (t2 != 0.0) & m2
    pd2 = pa_ref[...] - pb_ref[...]
    l2 = jnp.sum(jnp.where(w2, jnp.log(1.0 + jnp.exp(-t2 * pd2)), 0.0))
    n2 = jnp.sum(w2.astype(jnp.float32))
    out_ref[0, 0] = (l2 + l1) / (n2 + n1)


_loss_call = pl.pallas_call(
    _loss_body,
    out_specs=pl.BlockSpec(memory_space=pltpu.SMEM),
    out_shape=jax.ShapeDtypeStruct((1, 1), jnp.float32),
)


def kernel(pred_depth, gt_depth, tgt_valid_weight):
    bits = lax.bitcast_convert_type(tgt_valid_weight, jnp.int32).reshape(_B, _ROWS, 128)
    dst = _dst_call(bits).reshape(_B * _HW)
    gt1 = gt_depth.reshape(-1)
    pr1 = pred_depth.reshape(-1)
    outs = _sc_calls()(dst, jnp.asarray(_GIDX), jnp.asarray(_PA4),
                       jnp.asarray(_PB4), gt1, pr1)
    gtiv, priv, gtv, prv = [o.reshape(_B, _KP) for o in outs[:4]]
    ga, gb, pa, pb = [o.reshape(_B, _NPP) for o in outs[4:]]
    total = _loss_call(gtiv, priv, gtv, prv, ga, gb, pa, pb)
    return total[0, 0]
